# baseline (speedup 1.0000x reference)
import functools

import jax
import jax.numpy as jnp
from jax import lax
from jax.experimental import pallas as pl
from jax.experimental.pallas import tpu as pltpu
from jax.experimental.pallas import tpu_sc as plsc

T = 8
NCLIP = 8
C = 512
HW = 784
G = 4
GC = C // G
NW = 32
K = 8
NB = 7


PBLK = 112


def _pool_coef_body(x_ref, cw_ref, ww0_ref, ww1_ref, fcw_ref, fcb_ref,
                    lastw_ref, lastb_ref, convb_ref, wcb_ref,
                    pooled_ref, idxw_ref, coefw_ref):
    i = pl.program_id(0)

    @pl.when(i == 0)
    def _():
        pooled_ref[...] = jnp.zeros_like(pooled_ref)

    pooled_ref[...] += jnp.sum(x_ref[...], axis=0)

    @pl.when(i == HW // PBLK - 1)
    def _():
        _coef_math(pooled_ref, cw_ref, ww0_ref, ww1_ref, fcw_ref, fcb_ref,
                   lastw_ref, lastb_ref, convb_ref, wcb_ref,
                   idxw_ref, coefw_ref)


def _coef_math(pooled_ref, cw_ref, ww0_ref, ww1_ref, fcw_ref, fcb_ref,
               lastw_ref, lastb_ref, convb_ref, wcb_ref, idxw_ref, coefw_ref):
    P = pooled_ref[...]
    wall9 = jnp.concatenate(
        [cw_ref[...], ww0_ref[...], ww1_ref[...]], axis=1) * (1.0 / HW)
    M = jnp.dot(P, wall9, preferred_element_type=jnp.float32)

    ri = lax.broadcasted_iota(jnp.int32, (64, 64), 0)
    rj = lax.broadcasted_iota(jnp.int32, (64, 64), 1)
    sm = ((rj == ri - 1) & (ri % 8 != 0)).astype(jnp.float32)
    sp = ((rj == ri + 1) & (ri % 8 != 7)).astype(jnp.float32)
    Md = jnp.dot(sm, M, preferred_element_type=jnp.float32)
    Mu = jnp.dot(sp, M, preferred_element_type=jnp.float32)

    conv_b = convb_ref[0:1, 0:1]
    wconv_b0 = wcb_ref[0:1, 0:1]
    wconv_b1 = wcb_ref[1:2, 0:1]

    xb = Md[:, 0:1] + M[:, 1:2] + Mu[:, 2:3] + conv_b
    xw0 = Md[:, 3:4] + M[:, 4:5] + Mu[:, 5:6] + wconv_b0
    xw1 = Md[:, 6:7] + M[:, 7:8] + Mu[:, 8:9] + wconv_b1
    xweight0 = 2.0 * jax.nn.sigmoid(xw0)
    xweight1 = 2.0 * jax.nn.sigmoid(xw1)

    fcw = fcw_ref[...]
    fcb = fcb_ref[...]
    lastw = lastw_ref[...]
    lastb = lastb_ref[...]
    zlist = []
    for n in range(8):
        xbn = xb[8 * n:8 * n + 8, :]
        yn = jax.nn.relu(jnp.dot(fcw, xbn, preferred_element_type=jnp.float32) + fcb)
        zn = jnp.dot(lastw, yn, preferred_element_type=jnp.float32) + lastb
        zlist.append(zn)
    z = jnp.concatenate(zlist, axis=0)
    z = 4.0 * (jax.nn.sigmoid(z) - 0.5)

    wi = lax.broadcasted_iota(jnp.int32, (NW, 16), 0)
    kj = lax.broadcasted_iota(jnp.int32, (NW, 16), 1)
    nw = wi // G
    gw = wi % G
    sgn = jnp.where(gw < 2, 1.0, -1.0)
    ez = (kj == 2 * nw + (gw % 2)).astype(jnp.float32) * sgn
    Bw = jnp.dot(ez, z, preferred_element_type=jnp.float32)

    Bf = jnp.floor(Bw)
    b0 = Bf.astype(jnp.int32)
    w0 = 1.0 - (Bw - Bf)
    w1 = Bw - Bf

    xwcat = jnp.concatenate([xweight0, xweight1], axis=0)
    tj = kj % 8
    xwf = jnp.zeros((NW, 16), jnp.float32)
    ki = lax.broadcasted_iota(jnp.int32, (NW, 128), 1)
    for t in range(8):
        pt = (ki == 64 * (gw[:, 0:1] % 2) + 8 * nw[:, 0:1] + t).astype(jnp.float32)
        xt = jnp.dot(pt, xwcat, preferred_element_type=jnp.float32)
        xwf = xwf + xt * (tj == t).astype(jnp.float32)

    tap1 = (kj >= 8).astype(jnp.int32)
    t0 = tj + b0 + tap1
    valid = ((t0 >= 0) & (t0 < T)).astype(jnp.float32)
    idxw_ref[...] = jnp.clip(t0, 0, T - 1)
    wsel = jnp.where(kj < 8, w0, w1)
    coefw_ref[...] = xwf * wsel * valid


def _pool_coefs(xT, cw, ww0, ww1, fcw, fcb, lastw, lastb, convb, wcb):
    full = lambda shape: pl.BlockSpec(shape, lambda i: tuple(0 for _ in shape))
    _, idxw, coefw = pl.pallas_call(
        _pool_coef_body,
        grid=(HW // PBLK,),
        in_specs=[
            pl.BlockSpec((PBLK, 64, C), lambda i: (i, 0, 0)),
            full((C, 3)),
            full((C, 3)),
            full((C, 3)),
            full((8, 8)),
            full((8, 1)),
            full((2, 8)),
            full((2, 1)),
            full((1, 1)),
            full((2, 1)),
        ],
        out_specs=(
            full((64, C)),
            full((NW, 16)),
            full((NW, 16)),
        ),
        out_shape=(
            jax.ShapeDtypeStruct((64, C), jnp.float32),
            jax.ShapeDtypeStruct((NW, 16), jnp.int32),
            jax.ShapeDtypeStruct((NW, 16), jnp.float32),
        ),
    )(xT, cw, ww0, ww1, fcw, fcb, lastw, lastb, convb, wcb)
    return idxw, coefw


def _sc_body(nc, xT_hbm, idxw_hbm, coefw_hbm, out_hbm,
             idx_v, coef_v, *rest):
    wid = lax.axis_index("s") * nc + lax.axis_index("c")
    bufin = rest[0:NB]
    bufout = rest[NB:2 * NB]
    sg = rest[2 * NB:3 * NB]
    ss = rest[3 * NB:4 * NB]

    pltpu.sync_copy(idxw_hbm, idx_v)
    pltpu.sync_copy(coefw_hbm, coef_v)

    nb8 = pl.multiple_of(8 * (wid // G), 8)
    gb = pl.multiple_of(GC * (wid % G), GC)

    lane = lax.iota(jnp.int32, 16)
    iv = idx_v[wid, pl.ds(0, 16)]
    cv = coef_v[wid, pl.ds(0, 16)]
    r0 = [jnp.sum(jnp.where(lane == t, iv, 0)) for t in range(8)]
    r1 = [jnp.sum(jnp.where(lane == 8 + t, iv, 0)) for t in range(8)]
    c0 = [jnp.full((16,), jnp.sum(jnp.where(lane == t, cv, 0.0)), jnp.float32)
          for t in range(8)]
    c1 = [jnp.full((16,), jnp.sum(jnp.where(lane == 8 + t, cv, 0.0)), jnp.float32)
          for t in range(8)]

    def window(task):
        return (pl.ds(task * K, K), pl.ds(nb8, 8), pl.ds(gb, GC))

    def issue_gather(b, task):
        pltpu.make_async_copy(xT_hbm.at[window(task)], bufin[b], sg[b]).start()

    def wait_gather(b):
        pltpu.make_async_copy(xT_hbm.at[window(0)], bufin[b], sg[b]).wait()

    u = r0 + [r1[7]]

    def compute(b):
        A, O = bufin[b], bufout[b]

        def kbody(k, carry):
            for l in range(GC // 16):
                sl = pl.ds(l * 16, 16)
                v = [A[k, u[t], sl] for t in range(9)]
                for t in range(8):
                    O[k, t, sl] = c0[t] * v[t] + c1[t] * v[t + 1]
            return carry

        lax.fori_loop(0, K, kbody, 0)

    def issue_store(b, task):
        pltpu.make_async_copy(bufout[b], out_hbm.at[window(task)], ss[b]).start()

    def wait_store(b, task):
        pltpu.make_async_copy(bufout[b], out_hbm.at[window(task)], ss[b]).wait()

    ntask = HW // K
    nround = ntask // NB

    for b in range(NB):
        issue_gather(b, b)

    def round_body(r, carry):
        for b in range(NB):
            t = r * NB + b
            wait_gather(b)

            @pl.when(r > 0)
            def _():
                wait_store(b, t - NB)

            compute(b)
            issue_store(b, t)

            @pl.when(r < nround - 1)
            def _():
                issue_gather(b, t + NB)

        return carry

    lax.fori_loop(0, nround, round_body, 0)

    for b in range(NB):
        wait_store(b, (nround - 1) * NB + b)


def _gather_lerp(xT, idxw, coefw):
    info = plsc.get_sparse_core_info()
    mesh = plsc.VectorSubcoreMesh(core_axis_name="c", subcore_axis_name="s")
    fn = pl.kernel(
        functools.partial(_sc_body, info.num_cores),
        out_type=jax.ShapeDtypeStruct((HW, 64, C), jnp.float32),
        mesh=mesh,
        scratch_types=(
            [
                pltpu.VMEM((NW, 16), jnp.int32),
                pltpu.VMEM((NW, 16), jnp.float32),
            ]
            + [pltpu.VMEM((K, 8, GC), jnp.float32) for _ in range(2 * NB)]
            + [pltpu.SemaphoreType.DMA for _ in range(2 * NB)]
        ),
        compiler_params=pltpu.CompilerParams(needs_layout_passes=False),
    )
    return fn(xT, idxw, coefw)


def kernel(x, conv_w, conv_b, fc_w, fc_b, last_w, last_b, wconv_w, wconv_b):
    nt, c, h, w = x.shape
    xT = jnp.transpose(x, (2, 3, 0, 1)).reshape(HW, nt, c)

    idxw, coefw = _pool_coefs(
        xT, conv_w[0], wconv_w[0], wconv_w[1], fc_w, fc_b.reshape(8, 1),
        last_w, last_b.reshape(2, 1), conv_b.reshape(1, 1), wconv_b.reshape(2, 1))

    outT = _gather_lerp(xT, idxw, coefw)
    return jnp.transpose(outT.reshape(h, w, nt, c), (2, 3, 0, 1))


# speedup vs baseline: 1.0060x; 1.0060x over previous
import functools

import jax
import jax.numpy as jnp
from jax import lax
from jax.experimental import pallas as pl
from jax.experimental.pallas import tpu as pltpu
from jax.experimental.pallas import tpu_sc as plsc

T = 8
NCLIP = 8
C = 512
HW = 784
G = 4
GC = C // G
NW = 32
K = 8
NB = 7


PBLK = 56


def _pool_coef_body(x_ref, cw_ref, ww0_ref, ww1_ref, fcw_ref, fcb_ref,
                    lastw_ref, lastb_ref, convb_ref, wcb_ref,
                    pooled_ref, idxw_ref, coefw_ref):
    i = pl.program_id(0)

    @pl.when(i == 0)
    def _():
        pooled_ref[...] = jnp.zeros_like(pooled_ref)

    pooled_ref[...] += jnp.sum(x_ref[...], axis=0)

    @pl.when(i == HW // PBLK - 1)
    def _():
        _coef_math(pooled_ref, cw_ref, ww0_ref, ww1_ref, fcw_ref, fcb_ref,
                   lastw_ref, lastb_ref, convb_ref, wcb_ref,
                   idxw_ref, coefw_ref)


def _coef_math(pooled_ref, cw_ref, ww0_ref, ww1_ref, fcw_ref, fcb_ref,
               lastw_ref, lastb_ref, convb_ref, wcb_ref, idxw_ref, coefw_ref):
    P = pooled_ref[...]
    wall9 = jnp.concatenate(
        [cw_ref[...], ww0_ref[...], ww1_ref[...]], axis=1) * (1.0 / HW)
    M = jnp.dot(P, wall9, preferred_element_type=jnp.float32)

    ri = lax.broadcasted_iota(jnp.int32, (64, 64), 0)
    rj = lax.broadcasted_iota(jnp.int32, (64, 64), 1)
    sm = ((rj == ri - 1) & (ri % 8 != 0)).astype(jnp.float32)
    sp = ((rj == ri + 1) & (ri % 8 != 7)).astype(jnp.float32)
    Md = jnp.dot(sm, M, preferred_element_type=jnp.float32)
    Mu = jnp.dot(sp, M, preferred_element_type=jnp.float32)

    conv_b = convb_ref[0:1, 0:1]
    wconv_b0 = wcb_ref[0:1, 0:1]
    wconv_b1 = wcb_ref[1:2, 0:1]

    xb = Md[:, 0:1] + M[:, 1:2] + Mu[:, 2:3] + conv_b
    xw0 = Md[:, 3:4] + M[:, 4:5] + Mu[:, 5:6] + wconv_b0
    xw1 = Md[:, 6:7] + M[:, 7:8] + Mu[:, 8:9] + wconv_b1
    xweight0 = 2.0 * jax.nn.sigmoid(xw0)
    xweight1 = 2.0 * jax.nn.sigmoid(xw1)

    fcw = fcw_ref[...]
    fcb = fcb_ref[...]
    lastw = lastw_ref[...]
    lastb = lastb_ref[...]
    zlist = []
    for n in range(8):
        xbn = xb[8 * n:8 * n + 8, :]
        yn = jax.nn.relu(jnp.dot(fcw, xbn, preferred_element_type=jnp.float32) + fcb)
        zn = jnp.dot(lastw, yn, preferred_element_type=jnp.float32) + lastb
        zlist.append(zn)
    z = jnp.concatenate(zlist, axis=0)
    z = 4.0 * (jax.nn.sigmoid(z) - 0.5)

    wi = lax.broadcasted_iota(jnp.int32, (NW, 16), 0)
    kj = lax.broadcasted_iota(jnp.int32, (NW, 16), 1)
    nw = wi // G
    gw = wi % G
    sgn = jnp.where(gw < 2, 1.0, -1.0)
    ez = (kj == 2 * nw + (gw % 2)).astype(jnp.float32) * sgn
    Bw = jnp.dot(ez, z, preferred_element_type=jnp.float32)

    Bf = jnp.floor(Bw)
    b0 = Bf.astype(jnp.int32)
    w0 = 1.0 - (Bw - Bf)
    w1 = Bw - Bf

    xwcat = jnp.concatenate([xweight0, xweight1], axis=0)
    tj = kj % 8
    xwf = jnp.zeros((NW, 16), jnp.float32)
    ki = lax.broadcasted_iota(jnp.int32, (NW, 128), 1)
    for t in range(8):
        pt = (ki == 64 * (gw[:, 0:1] % 2) + 8 * nw[:, 0:1] + t).astype(jnp.float32)
        xt = jnp.dot(pt, xwcat, preferred_element_type=jnp.float32)
        xwf = xwf + xt * (tj == t).astype(jnp.float32)

    tap1 = (kj >= 8).astype(jnp.int32)
    t0 = tj + b0 + tap1
    valid = ((t0 >= 0) & (t0 < T)).astype(jnp.float32)
    idxw_ref[...] = jnp.clip(t0, 0, T - 1)
    wsel = jnp.where(kj < 8, w0, w1)
    coefw_ref[...] = xwf * wsel * valid


def _pool_coefs(xT, cw, ww0, ww1, fcw, fcb, lastw, lastb, convb, wcb):
    full = lambda shape: pl.BlockSpec(shape, lambda i: tuple(0 for _ in shape))
    _, idxw, coefw = pl.pallas_call(
        _pool_coef_body,
        grid=(HW // PBLK,),
        in_specs=[
            pl.BlockSpec((PBLK, 64, C), lambda i: (i, 0, 0)),
            full((C, 3)),
            full((C, 3)),
            full((C, 3)),
            full((8, 8)),
            full((8, 1)),
            full((2, 8)),
            full((2, 1)),
            full((1, 1)),
            full((2, 1)),
        ],
        out_specs=(
            full((64, C)),
            full((NW, 16)),
            full((NW, 16)),
        ),
        out_shape=(
            jax.ShapeDtypeStruct((64, C), jnp.float32),
            jax.ShapeDtypeStruct((NW, 16), jnp.int32),
            jax.ShapeDtypeStruct((NW, 16), jnp.float32),
        ),
    )(xT, cw, ww0, ww1, fcw, fcb, lastw, lastb, convb, wcb)
    return idxw, coefw


def _sc_body(nc, xT_hbm, idxw_hbm, coefw_hbm, out_hbm,
             idx_v, coef_v, *rest):
    wid = lax.axis_index("s") * nc + lax.axis_index("c")
    bufin = rest[0:NB]
    bufout = rest[NB:2 * NB]
    sg = rest[2 * NB:3 * NB]
    ss = rest[3 * NB:4 * NB]

    nb8 = pl.multiple_of(8 * (wid // G), 8)
    gb = pl.multiple_of(GC * (wid % G), GC)

    def window0(task):
        return (pl.ds(task * K, K), pl.ds(nb8, 8), pl.ds(gb, GC))

    for b in range(NB):
        pltpu.make_async_copy(xT_hbm.at[window0(b)], bufin[b], sg[b]).start()

    pltpu.sync_copy(idxw_hbm, idx_v)
    pltpu.sync_copy(coefw_hbm, coef_v)

    lane = lax.iota(jnp.int32, 16)
    iv = idx_v[wid, pl.ds(0, 16)]
    cv = coef_v[wid, pl.ds(0, 16)]
    r0 = [jnp.sum(jnp.where(lane == t, iv, 0)) for t in range(8)]
    r1 = [jnp.sum(jnp.where(lane == 8 + t, iv, 0)) for t in range(8)]
    c0 = [jnp.full((16,), jnp.sum(jnp.where(lane == t, cv, 0.0)), jnp.float32)
          for t in range(8)]
    c1 = [jnp.full((16,), jnp.sum(jnp.where(lane == 8 + t, cv, 0.0)), jnp.float32)
          for t in range(8)]

    def window(task):
        return (pl.ds(task * K, K), pl.ds(nb8, 8), pl.ds(gb, GC))

    def issue_gather(b, task):
        pltpu.make_async_copy(xT_hbm.at[window(task)], bufin[b], sg[b]).start()

    def wait_gather(b):
        pltpu.make_async_copy(xT_hbm.at[window(0)], bufin[b], sg[b]).wait()

    u = r0 + [r1[7]]

    def compute(b):
        A, O = bufin[b], bufout[b]

        def kbody(k, carry):
            for l in range(GC // 16):
                sl = pl.ds(l * 16, 16)
                v = [A[k, u[t], sl] for t in range(9)]
                for t in range(8):
                    O[k, t, sl] = c0[t] * v[t] + c1[t] * v[t + 1]
            return carry

        lax.fori_loop(0, K, kbody, 0)

    def issue_store(b, task):
        pltpu.make_async_copy(bufout[b], out_hbm.at[window(task)], ss[b]).start()

    def wait_store(b, task):
        pltpu.make_async_copy(bufout[b], out_hbm.at[window(task)], ss[b]).wait()

    ntask = HW // K
    nround = ntask // NB

    def round_body(r, carry):
        for b in range(NB):
            t = r * NB + b
            wait_gather(b)

            @pl.when(r > 0)
            def _():
                wait_store(b, t - NB)

            compute(b)
            issue_store(b, t)

            @pl.when(r < nround - 1)
            def _():
                issue_gather(b, t + NB)

        return carry

    lax.fori_loop(0, nround, round_body, 0)

    for b in range(NB):
        wait_store(b, (nround - 1) * NB + b)


def _gather_lerp(xT, idxw, coefw):
    info = plsc.get_sparse_core_info()
    mesh = plsc.VectorSubcoreMesh(core_axis_name="c", subcore_axis_name="s")
    fn = pl.kernel(
        functools.partial(_sc_body, info.num_cores),
        out_type=jax.ShapeDtypeStruct((HW, 64, C), jnp.float32),
        mesh=mesh,
        scratch_types=(
            [
                pltpu.VMEM((NW, 16), jnp.int32),
                pltpu.VMEM((NW, 16), jnp.float32),
            ]
            + [pltpu.VMEM((K, 8, GC), jnp.float32) for _ in range(2 * NB)]
            + [pltpu.SemaphoreType.DMA for _ in range(2 * NB)]
        ),
        compiler_params=pltpu.CompilerParams(needs_layout_passes=False),
    )
    return fn(xT, idxw, coefw)


def kernel(x, conv_w, conv_b, fc_w, fc_b, last_w, last_b, wconv_w, wconv_b):
    nt, c, h, w = x.shape
    xT = jnp.transpose(x, (2, 3, 0, 1)).reshape(HW, nt, c)

    idxw, coefw = _pool_coefs(
        xT, conv_w[0], wconv_w[0], wconv_w[1], fc_w, fc_b.reshape(8, 1),
        last_w, last_b.reshape(2, 1), conv_b.reshape(1, 1), wconv_b.reshape(2, 1))

    outT = _gather_lerp(xT, idxw, coefw)
    return jnp.transpose(outT.reshape(h, w, nt, c), (2, 3, 0, 1))


# speedup vs baseline: 1.0066x; 1.0006x over previous
import functools

import jax
import jax.numpy as jnp
from jax import lax
from jax.experimental import pallas as pl
from jax.experimental.pallas import tpu as pltpu
from jax.experimental.pallas import tpu_sc as plsc

T = 8
NCLIP = 8
C = 512
HW = 784
G = 4
GC = C // G
NW = 32
K = 8
NB = 7


PBLK = 56


def _pool_coef_body(x_ref, cw_ref, ww0_ref, ww1_ref, fcw_ref, fcb_ref,
                    lastw_ref, lastb_ref, convb_ref, wcb_ref,
                    pooled_ref, idxw_ref, coefw_ref):
    i = pl.program_id(0)

    @pl.when(i == 0)
    def _():
        pooled_ref[...] = jnp.zeros_like(pooled_ref)

    pooled_ref[...] += jnp.sum(x_ref[...], axis=0)

    @pl.when(i == HW // PBLK - 1)
    def _():
        _coef_math(pooled_ref, cw_ref, ww0_ref, ww1_ref, fcw_ref, fcb_ref,
                   lastw_ref, lastb_ref, convb_ref, wcb_ref,
                   idxw_ref, coefw_ref)


def _coef_math(pooled_ref, cw_ref, ww0_ref, ww1_ref, fcw_ref, fcb_ref,
               lastw_ref, lastb_ref, convb_ref, wcb_ref, idxw_ref, coefw_ref):
    P = pooled_ref[...]
    wall9 = jnp.concatenate(
        [cw_ref[...], ww0_ref[...], ww1_ref[...]], axis=1) * (1.0 / HW)
    M = jnp.dot(P, wall9, preferred_element_type=jnp.float32)

    ri = lax.broadcasted_iota(jnp.int32, (64, 64), 0)
    rj = lax.broadcasted_iota(jnp.int32, (64, 64), 1)
    sm = ((rj == ri - 1) & (ri % 8 != 0)).astype(jnp.float32)
    sp = ((rj == ri + 1) & (ri % 8 != 7)).astype(jnp.float32)
    Md = jnp.dot(sm, M, preferred_element_type=jnp.float32)
    Mu = jnp.dot(sp, M, preferred_element_type=jnp.float32)

    conv_b = convb_ref[0:1, 0:1]
    wconv_b0 = wcb_ref[0:1, 0:1]
    wconv_b1 = wcb_ref[1:2, 0:1]

    xb = Md[:, 0:1] + M[:, 1:2] + Mu[:, 2:3] + conv_b
    xw0 = Md[:, 3:4] + M[:, 4:5] + Mu[:, 5:6] + wconv_b0
    xw1 = Md[:, 6:7] + M[:, 7:8] + Mu[:, 8:9] + wconv_b1
    xweight0 = 2.0 * jax.nn.sigmoid(xw0)
    xweight1 = 2.0 * jax.nn.sigmoid(xw1)

    fcw = fcw_ref[...]
    fcb = fcb_ref[...]
    lastw = lastw_ref[...]
    lastb = lastb_ref[...]
    zlist = []
    for n in range(8):
        xbn = xb[8 * n:8 * n + 8, :]
        yn = jax.nn.relu(jnp.dot(fcw, xbn, preferred_element_type=jnp.float32) + fcb)
        zn = jnp.dot(lastw, yn, preferred_element_type=jnp.float32) + lastb
        zlist.append(zn)
    z = jnp.concatenate(zlist, axis=0)
    z = 4.0 * (jax.nn.sigmoid(z) - 0.5)

    wi = lax.broadcasted_iota(jnp.int32, (NW, 16), 0)
    kj = lax.broadcasted_iota(jnp.int32, (NW, 16), 1)
    nw = wi // G
    gw = wi % G
    sgn = jnp.where(gw < 2, 1.0, -1.0)
    ez = (kj == 2 * nw + (gw % 2)).astype(jnp.float32) * sgn
    Bw = jnp.dot(ez, z, preferred_element_type=jnp.float32)

    Bf = jnp.floor(Bw)
    b0 = Bf.astype(jnp.int32)
    w0 = 1.0 - (Bw - Bf)
    w1 = Bw - Bf

    xwcat = jnp.concatenate([xweight0, xweight1], axis=0)
    tj = kj % 8
    xwf = jnp.zeros((NW, 16), jnp.float32)
    ki = lax.broadcasted_iota(jnp.int32, (NW, 128), 1)
    for t in range(8):
        pt = (ki == 64 * (gw[:, 0:1] % 2) + 8 * nw[:, 0:1] + t).astype(jnp.float32)
        xt = jnp.dot(pt, xwcat, preferred_element_type=jnp.float32)
        xwf = xwf + xt * (tj == t).astype(jnp.float32)

    tap1 = (kj >= 8).astype(jnp.int32)
    t0 = tj + b0 + tap1
    valid = ((t0 >= 0) & (t0 < T)).astype(jnp.float32)
    idxw_ref[...] = jnp.clip(t0, 0, T - 1)
    wsel = jnp.where(kj < 8, w0, w1)
    coefw_ref[...] = xwf * wsel * valid


def _pool_coefs(xT, cw, ww0, ww1, fcw, fcb, lastw, lastb, convb, wcb):
    full = lambda shape: pl.BlockSpec(shape, lambda i: tuple(0 for _ in shape))
    _, idxw, coefw = pl.pallas_call(
        _pool_coef_body,
        grid=(HW // PBLK,),
        in_specs=[
            pl.BlockSpec((PBLK, 64, C), lambda i: (i, 0, 0)),
            full((C, 3)),
            full((C, 3)),
            full((C, 3)),
            full((8, 8)),
            full((8, 1)),
            full((2, 8)),
            full((2, 1)),
            full((1, 1)),
            full((2, 1)),
        ],
        out_specs=(
            full((64, C)),
            full((NW, 16)),
            full((NW, 16)),
        ),
        out_shape=(
            jax.ShapeDtypeStruct((64, C), jnp.float32),
            jax.ShapeDtypeStruct((NW, 16), jnp.int32),
            jax.ShapeDtypeStruct((NW, 16), jnp.float32),
        ),
    )(xT, cw, ww0, ww1, fcw, fcb, lastw, lastb, convb, wcb)
    return idxw, coefw


def _sc_body(nc, xT_hbm, idxw_hbm, coefw_hbm, out_hbm,
             idx_v, coef_v, *rest):
    wid = lax.axis_index("s") * nc + lax.axis_index("c")
    bufin = rest[0:NB]
    bufout = rest[NB:2 * NB]
    sg = rest[2 * NB:3 * NB]
    ss = rest[3 * NB:4 * NB]

    nb8 = pl.multiple_of(8 * (wid // G), 8)
    gb = pl.multiple_of(GC * (wid % G), GC)

    def window(task):
        return (pl.ds(task * K, K), pl.ds(nb8, 8), pl.ds(gb, GC))

    for b in range(NB):
        pltpu.make_async_copy(xT_hbm.at[window(b)], bufin[b], sg[b]).start()

    pltpu.sync_copy(idxw_hbm, idx_v)
    pltpu.sync_copy(coefw_hbm, coef_v)

    lane = lax.iota(jnp.int32, 16)
    iv = idx_v[wid, pl.ds(0, 16)]
    cv = coef_v[wid, pl.ds(0, 16)]
    r0 = [jnp.sum(jnp.where(lane == t, iv, 0)) for t in range(8)]
    r1 = [jnp.sum(jnp.where(lane == 8 + t, iv, 0)) for t in range(8)]
    c0 = [jnp.full((16,), jnp.sum(jnp.where(lane == t, cv, 0.0)), jnp.float32)
          for t in range(8)]
    c1 = [jnp.full((16,), jnp.sum(jnp.where(lane == 8 + t, cv, 0.0)), jnp.float32)
          for t in range(8)]

    def issue_gather(b, task):
        pltpu.make_async_copy(xT_hbm.at[window(task)], bufin[b], sg[b]).start()

    def wait_gather(b):
        pltpu.make_async_copy(xT_hbm.at[window(0)], bufin[b], sg[b]).wait()

    u = r0 + [r1[7]]

    def compute(b):
        A, O = bufin[b], bufout[b]

        def kbody(k, carry):
            for l in range(GC // 16):
                sl = pl.ds(l * 16, 16)
                v = [A[k, u[t], sl] for t in range(9)]
                for t in range(8):
                    O[k, t, sl] = c0[t] * v[t] + c1[t] * v[t + 1]
            return carry

        lax.fori_loop(0, K, kbody, 0)

    def issue_store(b, task):
        pltpu.make_async_copy(bufout[b], out_hbm.at[window(task)], ss[b]).start()

    def wait_store(b, task):
        pltpu.make_async_copy(bufout[b], out_hbm.at[window(task)], ss[b]).wait()

    ntask = HW // K
    nround = ntask // NB

    def round_body(r, carry):
        for b in range(NB):
            t = r * NB + b
            wait_gather(b)

            @pl.when(r > 0)
            def _():
                wait_store(b, t - NB)

            compute(b)
            issue_store(b, t)

            @pl.when(r < nround - 1)
            def _():
                issue_gather(b, t + NB)

        return carry

    lax.fori_loop(0, nround, round_body, 0)

    for b in range(NB):
        wait_store(b, (nround - 1) * NB + b)


def _gather_lerp(xT, idxw, coefw):
    info = plsc.get_sparse_core_info()
    mesh = plsc.VectorSubcoreMesh(core_axis_name="c", subcore_axis_name="s")
    fn = pl.kernel(
        functools.partial(_sc_body, info.num_cores),
        out_type=jax.ShapeDtypeStruct((HW, 64, C), jnp.float32),
        mesh=mesh,
        scratch_types=(
            [
                pltpu.VMEM((NW, 16), jnp.int32),
                pltpu.VMEM((NW, 16), jnp.float32),
            ]
            + [pltpu.VMEM((K, 8, GC), jnp.float32) for _ in range(2 * NB)]
            + [pltpu.SemaphoreType.DMA for _ in range(2 * NB)]
        ),
        compiler_params=pltpu.CompilerParams(needs_layout_passes=False),
    )
    return fn(xT, idxw, coefw)


def kernel(x, conv_w, conv_b, fc_w, fc_b, last_w, last_b, wconv_w, wconv_b):
    nt, c, h, w = x.shape
    xT = jnp.transpose(x, (2, 3, 0, 1)).reshape(HW, nt, c)

    idxw, coefw = _pool_coefs(
        xT, conv_w[0], wconv_w[0], wconv_w[1], fc_w, fc_b.reshape(8, 1),
        last_w, last_b.reshape(2, 1), conv_b.reshape(1, 1), wconv_b.reshape(2, 1))

    outT = _gather_lerp(xT, idxw, coefw)
    return jnp.transpose(outT.reshape(h, w, nt, c), (2, 3, 0, 1))
